# baseline (speedup 1.0000x reference)
import functools

import jax
import jax.numpy as jnp
from jax import lax
from jax.experimental import pallas as pl
from jax.experimental.pallas import tpu as pltpu
from jax.experimental.pallas import tpu_sc as plsc

N_NODES = 10000
D = 128
NC = 2
NS = 16
NW = NC * NS
EB = 128
NBUF = 2
NIDX = 4
S_ROWS = 10112
SROWS_T = S_ROWS // NS


def _make_seg_sum(cpt):
    mesh = plsc.VectorSubcoreMesh(core_axis_name="c", subcore_axis_name="s")

    @functools.partial(
        pl.kernel,
        mesh=mesh,
        out_type=(
            jax.ShapeDtypeStruct((NC, S_ROWS, D), jnp.float32),
            jax.ShapeDtypeStruct((NW, cpt, EB, D), jnp.float32),
        ),
        scratch_types=[
            pltpu.VMEM((NIDX, EB), jnp.int32),
            pltpu.VMEM((NBUF, EB, D), jnp.float32),
            pltpu.VMEM_SHARED((S_ROWS, D), jnp.float32),
            pltpu.SemaphoreType.DMA((NIDX,)),
            pltpu.SemaphoreType.DMA((NBUF,)),
        ],
    )
    def seg_sum(src_hbm, dst_hbm, x_hbm, zeros_hbm, out_hbm, m_hbm,
                islot, rows, S, isem, rsem):
        c = lax.axis_index("c")
        s = lax.axis_index("s")
        wid = s * NC + c

        def fire_idx(idx_hbm, g, sl):
            pltpu.async_copy(idx_hbm.at[wid, g], islot.at[sl], isem.at[sl])

        def wait_idx(idx_hbm, g, sl):
            pltpu.make_async_copy(
                idx_hbm.at[wid, g], islot.at[sl], isem.at[sl]).wait()

        def fire_gather(sl, b):
            pltpu.async_copy(S.at[islot.at[sl]], rows.at[b], rsem.at[b])

        def wait_gather(sl, b):
            pltpu.make_async_copy(
                S.at[islot.at[sl]], rows.at[b], rsem.at[b]).wait()

        def fire_read(g, b):
            pltpu.async_copy(m_hbm.at[wid, g], rows.at[b], rsem.at[b])

        def wait_read(g, b):
            pltpu.make_async_copy(
                m_hbm.at[wid, g], rows.at[b], rsem.at[b]).wait()

        pltpu.sync_copy(x_hbm.at[pl.ds(s * SROWS_T, SROWS_T)],
                        S.at[pl.ds(s * SROWS_T, SROWS_T)])
        fire_idx(src_hbm, 0, 0)
        fire_idx(src_hbm, 1, 1)
        plsc.subcore_barrier()
        wait_idx(src_hbm, 0, 0)
        fire_gather(0, 0)

        def g_group(gi, carry):
            for u in range(NIDX):
                g = gi * NIDX + u
                b = u % NBUF
                nsl = (u + 1) % NIDX

                @pl.when(g + 1 < cpt)
                def _():
                    wait_idx(src_hbm, g + 1, nsl)
                    fire_gather(nsl, 1 - b)

                @pl.when(g + 2 < cpt)
                def _():
                    fire_idx(src_hbm, g + 2, (u + 2) % NIDX)

                wait_gather(u, b)
                pltpu.sync_copy(rows.at[b], m_hbm.at[wid, g])
            return carry

        lax.fori_loop(0, cpt // NIDX, g_group, 0)

        plsc.subcore_barrier()
        pltpu.sync_copy(zeros_hbm, S.at[pl.ds(s * SROWS_T, SROWS_T)])
        fire_idx(dst_hbm, 0, 0)
        fire_idx(dst_hbm, 1, 1)
        fire_read(0, 0)
        plsc.subcore_barrier()
        wait_idx(dst_hbm, 0, 0)

        def s_group(gi, carry):
            for u in range(NIDX):
                g = gi * NIDX + u
                b = u % NBUF
                nsl = (u + 1) % NIDX

                @pl.when(g + 1 < cpt)
                def _():
                    wait_idx(dst_hbm, g + 1, nsl)
                    fire_read(g + 1, 1 - b)

                @pl.when(g + 2 < cpt)
                def _():
                    fire_idx(dst_hbm, g + 2, (u + 2) % NIDX)

                wait_read(g, b)
                pltpu.sync_copy(rows.at[b], S.at[islot.at[u]], add=True)
            return carry

        lax.fori_loop(0, cpt // NIDX, s_group, 0)
        plsc.subcore_barrier()

        pltpu.sync_copy(S.at[pl.ds(s * SROWS_T, SROWS_T)],
                        out_hbm.at[c, pl.ds(s * SROWS_T, SROWS_T)])

    return seg_sum


def _mlp_kernel(parts_ref, x_ref, eps_ref,
                w1_ref, b1_ref, g1_ref, bt1_ref,
                w2_ref, b2_ref, g2_ref, bt2_ref, out_ref):
    n = x_ref.shape[0]
    out = (parts_ref[0, :n] + parts_ref[1, :n]
           + (1.0 + eps_ref[0, 0]) * x_ref[...])
    h = jnp.dot(out, w1_ref[...], preferred_element_type=jnp.float32)
    h = h + b1_ref[...]
    mu = jnp.mean(h, axis=0, keepdims=True)
    var = jnp.mean((h - mu) ** 2, axis=0, keepdims=True)
    h = (h - mu) * lax.rsqrt(var + 1e-5) * g1_ref[...] + bt1_ref[...]
    h = jnp.maximum(h, 0.0)
    h = jnp.dot(h, w2_ref[...], preferred_element_type=jnp.float32)
    h = h + b2_ref[...]
    mu = jnp.mean(h, axis=0, keepdims=True)
    var = jnp.mean((h - mu) ** 2, axis=0, keepdims=True)
    h = (h - mu) * lax.rsqrt(var + 1e-5) * g2_ref[...] + bt2_ref[...]
    out_ref[...] = jnp.maximum(h, 0.0)


def kernel(x, edge_index, eps, W1, b1, g1, bt1, W2, b2, g2, bt2):
    n, d = x.shape
    e = edge_index.shape[1]
    quantum = NW * EB * NIDX
    e_pad = quantum * (-(-e // quantum))
    cpt = e_pad // (NW * EB)
    src = edge_index[0]
    dst = edge_index[1]
    pad = e_pad - e
    if pad:
        fill = jnp.arange(pad, dtype=jnp.int32)
        src = jnp.concatenate([src, (fill * 97) % n])
        dst = jnp.concatenate([dst, n + (fill % (S_ROWS - n))])
    src = src.reshape(NW, cpt, EB)
    dst = dst.reshape(NW, cpt, EB)
    x_pad = jnp.concatenate([x, jnp.zeros((S_ROWS - n, d), jnp.float32)])
    zeros = jnp.zeros((SROWS_T, d), jnp.float32)

    parts, _ = _make_seg_sum(cpt)(src, dst, x_pad, zeros)

    out = pl.pallas_call(
        _mlp_kernel,
        out_shape=jax.ShapeDtypeStruct((n, d), jnp.float32),
    )(parts, x, eps.reshape(1, 1),
      W1, b1.reshape(1, d), g1.reshape(1, d), bt1.reshape(1, d),
      W2, b2.reshape(1, d), g2.reshape(1, d), bt2.reshape(1, d))
    return out
